# baseline (speedup 1.0000x reference)
import jax
import jax.numpy as jnp
from jax import lax
from jax.experimental import pallas as pl
from jax.experimental.pallas import tpu as pltpu

_CH = 64
_PF = 8
_NSLOT = 32


def _body(x_hbm, w_ref, o_hbm, mask_ref, xring, oring, xsems, osems):
    K, D = w_ref.shape
    B = x_hbm.shape[0]
    ntot = B // _CH

    z = w_ref[...] * 0.5
    zmax = jnp.max(z, axis=-1, keepdims=True)
    tau = zmax - 1.0
    for _ in range(10):
        r = jnp.maximum(z - tau, 0.0)
        g = jnp.sum(r * r, axis=-1, keepdims=True) - 1.0
        dg = 2.0 * jnp.sum(r, axis=-1, keepdims=True)
        tau = tau + g / dg
    mask_ref[...] = jnp.maximum(z - tau, 0.0) ** 2

    for p in range(_PF):
        pltpu.make_async_copy(
            x_hbm.at[pl.ds(p * _CH, _CH)], xring.at[p], xsems.at[p]).start()

    def step(g, carry):
        sx = lax.rem(g, _PF)
        so = lax.rem(g, _NSLOT)
        pltpu.make_async_copy(
            x_hbm.at[pl.ds(g * _CH, _CH)], xring.at[sx], xsems.at[sx]).wait()

        ocopy = pltpu.make_async_copy(
            oring.at[so], o_hbm.at[pl.ds(g * _CH, _CH)], osems.at[so])

        @pl.when(g >= _NSLOT)
        def _():
            ocopy.wait()

        xc = xring[sx]
        for n in range(K):
            oring[so, :, n, :] = xc * mask_ref[n, :]
        ocopy.start()

        @pl.when(g + _PF < ntot)
        def _():
            pltpu.make_async_copy(
                x_hbm.at[pl.ds((g + _PF) * _CH, _CH)],
                xring.at[sx], xsems.at[sx]).start()

        return carry

    lax.fori_loop(0, ntot, step, 0)

    for s in range(_NSLOT):
        pltpu.make_async_copy(
            oring.at[s], o_hbm.at[pl.ds(s * _CH, _CH)], osems.at[s]).wait()


@jax.jit
def kernel(x, W):
    B, D = x.shape
    K, _ = W.shape
    return pl.pallas_call(
        _body,
        grid=(1,),
        in_specs=[
            pl.BlockSpec(memory_space=pl.ANY),
            pl.BlockSpec((K, D), lambda i: (0, 0)),
        ],
        out_specs=pl.BlockSpec(memory_space=pl.ANY),
        out_shape=jax.ShapeDtypeStruct((B, K, D), x.dtype),
        scratch_shapes=[
            pltpu.VMEM((K, D), jnp.float32),
            pltpu.VMEM((_PF, _CH, D), jnp.float32),
            pltpu.VMEM((_NSLOT, _CH, K, D), jnp.float32),
            pltpu.SemaphoreType.DMA((_PF,)),
            pltpu.SemaphoreType.DMA((_NSLOT,)),
        ],
    )(x, W)


# speedup vs baseline: 1.1262x; 1.1262x over previous
import jax
import jax.numpy as jnp
from jax.experimental import pallas as pl
from jax.experimental.pallas import tpu as pltpu

_BLK = 2048
_NCH = 64
_CH = _BLK // _NCH
_NSLOT = 64


def _fused_body(x_ref, w_ref, o_hbm, mask_ref, buf_ref, sems):
    K, D = w_ref.shape
    i = pl.program_id(0)
    nsteps = pl.num_programs(0)

    @pl.when(i == 0)
    def _():
        z = w_ref[...] * 0.5
        zmax = jnp.max(z, axis=-1, keepdims=True)
        tau = zmax - 1.0
        for _ in range(10):
            r = jnp.maximum(z - tau, 0.0)
            g = jnp.sum(r * r, axis=-1, keepdims=True) - 1.0
            dg = 2.0 * jnp.sum(r, axis=-1, keepdims=True)
            tau = tau + g / dg
        mask_ref[...] = jnp.maximum(z - tau, 0.0) ** 2

    for c in range(_NCH):
        s = c % _NSLOT
        row0 = i * _BLK + c * _CH
        copy = pltpu.make_async_copy(
            buf_ref.at[s], o_hbm.at[pl.ds(row0, _CH)], sems.at[s])

        if c >= _NSLOT:
            copy.wait()
        else:
            @pl.when(i > 0)
            def _():
                copy.wait()

        xc = x_ref[pl.ds(c * _CH, _CH), :]
        for n in range(K):
            buf_ref[s, :, n, :] = xc * mask_ref[n, :]
        copy.start()

    @pl.when(i == nsteps - 1)
    def _():
        for c in range(_NCH - _NSLOT, _NCH):
            pltpu.make_async_copy(
                buf_ref.at[c % _NSLOT],
                o_hbm.at[pl.ds(i * _BLK + c * _CH, _CH)],
                sems.at[c % _NSLOT],
            ).wait()


@jax.jit
def kernel(x, W):
    B, D = x.shape
    K, _ = W.shape
    grid = (B // _BLK,)
    return pl.pallas_call(
        _fused_body,
        grid=grid,
        in_specs=[
            pl.BlockSpec((_BLK, D), lambda i: (i, 0)),
            pl.BlockSpec((K, D), lambda i: (0, 0)),
        ],
        out_specs=pl.BlockSpec(memory_space=pl.ANY),
        out_shape=jax.ShapeDtypeStruct((B, K, D), x.dtype),
        scratch_shapes=[
            pltpu.VMEM((K, D), jnp.float32),
            pltpu.VMEM((_NSLOT, _CH, K, D), jnp.float32),
            pltpu.SemaphoreType.DMA((_NSLOT,)),
        ],
    )(x, W)


# speedup vs baseline: 1.1283x; 1.0018x over previous
import jax
import jax.numpy as jnp
from jax.experimental import pallas as pl
from jax.experimental.pallas import tpu as pltpu

_BLK = 2048
_NCH = 32
_CH = _BLK // _NCH
_NSLOT = 32


def _fused_body(x_ref, w_ref, o_hbm, mask_ref, buf_ref, sems):
    K, D = w_ref.shape
    i = pl.program_id(0)
    nsteps = pl.num_programs(0)

    @pl.when(i == 0)
    def _():
        z = w_ref[...] * 0.5
        zmax = jnp.max(z, axis=-1, keepdims=True)
        tau = zmax - 1.0
        for _ in range(10):
            r = jnp.maximum(z - tau, 0.0)
            g = jnp.sum(r * r, axis=-1, keepdims=True) - 1.0
            dg = 2.0 * jnp.sum(r, axis=-1, keepdims=True)
            tau = tau + g / dg
        mask_ref[...] = jnp.maximum(z - tau, 0.0) ** 2

    for c in range(_NCH):
        s = c % _NSLOT
        row0 = i * _BLK + c * _CH
        copy = pltpu.make_async_copy(
            buf_ref.at[s], o_hbm.at[pl.ds(row0, _CH)], sems.at[s])

        if c >= _NSLOT:
            copy.wait()
        else:
            @pl.when(i > 0)
            def _():
                copy.wait()

        xc = x_ref[pl.ds(c * _CH, _CH), :]
        for n in range(K):
            buf_ref[s, :, n, :] = xc * mask_ref[n, :]
        copy.start()

    @pl.when(i == nsteps - 1)
    def _():
        for c in range(_NCH - _NSLOT, _NCH):
            pltpu.make_async_copy(
                buf_ref.at[c % _NSLOT],
                o_hbm.at[pl.ds(i * _BLK + c * _CH, _CH)],
                sems.at[c % _NSLOT],
            ).wait()


@jax.jit
def kernel(x, W):
    B, D = x.shape
    K, _ = W.shape
    grid = (B // _BLK,)
    return pl.pallas_call(
        _fused_body,
        grid=grid,
        in_specs=[
            pl.BlockSpec((_BLK, D), lambda i: (i, 0)),
            pl.BlockSpec((K, D), lambda i: (0, 0)),
        ],
        out_specs=pl.BlockSpec(memory_space=pl.ANY),
        out_shape=jax.ShapeDtypeStruct((B, K, D), x.dtype),
        scratch_shapes=[
            pltpu.VMEM((K, D), jnp.float32),
            pltpu.VMEM((_NSLOT, _CH, K, D), jnp.float32),
            pltpu.SemaphoreType.DMA((_NSLOT,)),
        ],
    )(x, W)
